# baseline (speedup 1.0000x reference)
import functools

import jax
import jax.numpy as jnp
from jax import lax
from jax.experimental import pallas as pl
from jax.experimental.pallas import tpu as pltpu
from jax.experimental.pallas import tpu_sc as plsc

NUM_USER = 20000
NUM_ITEM = 30000
N_NODES = NUM_USER + NUM_ITEM
DIM_FEAT = 128
DIM_LATENT = 64
HALF = DIM_LATENT // 2
N_EDGES = 800000
NUM_LAYER = 2

NC = 2
NS = 16
LANES = 16

K_EDGE = 80
E_PAD = 819200
CHUNKS_PER_TILE = E_PAD // (NS * K_EDGE)
SUPER = 64
N_SUPER = CHUNKS_PER_TILE // SUPER
N_PAD = 50048
ROWS_PER_TILE = N_PAD // NS
ZROWS = 136


def _mlp_body(f_ref, w1_ref, b1_ref, w2_ref, b2_ref, o_ref):
    h = jnp.dot(f_ref[...], w1_ref[...], preferred_element_type=jnp.float32)
    h = h + b1_ref[...]
    h = jnp.where(h >= 0, h, 0.01 * h)
    t = jnp.dot(h, w2_ref[...], preferred_element_type=jnp.float32)
    o_ref[...] = t + b2_ref[...]


def _norm_body(x_ref, oa_ref, ob_ref):
    x = x_ref[...]
    nrm = jnp.sqrt(jnp.sum(x * x, axis=1, keepdims=True))
    x = x / jnp.maximum(nrm, 1e-12)
    oa_ref[...] = x[:, :HALF]
    ob_ref[...] = x[:, HALF:]


def _cos_weight(sa, sb, ea, eb):
    dot = jnp.sum(sa * ea + sb * eb, axis=1, keepdims=True)
    n1 = jnp.sqrt(jnp.sum(sa * sa + sb * sb, axis=1, keepdims=True))
    n2 = jnp.sqrt(jnp.sum(ea * ea + eb * eb, axis=1, keepdims=True))
    return dot / (jnp.maximum(n1, 1e-8) * jnp.maximum(n2, 1e-8))


def _layer1_body(sa_ref, sb_ref, ea_ref, eb_ref, oa_ref, ob_ref):
    sa, sb = sa_ref[...], sb_ref[...]
    ea, eb = ea_ref[...], eb_ref[...]
    w = _cos_weight(sa, sb, ea, eb)
    oa_ref[...] = w * sa
    ob_ref[...] = w * sb


def _layer2_body(sa_ref, sb_ref, ea_ref, eb_ref, oa_ref, ob_ref):
    sa, sb = sa_ref[...], sb_ref[...]
    ea, eb = ea_ref[...], eb_ref[...]
    w = _cos_weight(sa, sb, ea, eb)
    oa_ref[...] = w * sa
    ob_ref[...] = w * sb


def _spmm_body(xa, xb, src2, dst2, vals2, outa, outb,
               src_v, dst_v, vals_v, rows_v, rows_v1, zbuf, acc,
               sem, sem1, ssem0, ssem1):
    cid = lax.axis_index("c")
    sid = lax.axis_index("s")

    def _zrow(i, c):
        z = jnp.zeros((LANES,), jnp.float32)
        zbuf[i, pl.ds(0, LANES)] = z
        zbuf[i, pl.ds(LANES, LANES)] = z
        return c
    lax.fori_loop(0, ZROWS, _zrow, 0)
    base0 = sid * ROWS_PER_TILE
    for i in range(ROWS_PER_TILE // ZROWS):
        pltpu.sync_copy(zbuf, acc.at[pl.ds(base0 + i * ZROWS, ZROWS)])
    plsc.subcore_barrier()
    del base0

    def run_half(x_h, out_h):
        row0 = sid * CHUNKS_PER_TILE
        bufs = (rows_v, rows_v1)
        sems = (sem, sem1)
        ssems = (ssem0, ssem1)

        def scale_chunk(j, buf):
            for g in range(K_EDGE // LANES):
                e0 = g * LANES
                v16 = vals_v[j, pl.ds(e0, LANES)]
                for l in range(LANES):
                    e = e0 + l
                    val = v16[l]
                    r0 = buf[e, pl.ds(0, LANES)]
                    r1 = buf[e, pl.ds(LANES, LANES)]
                    buf[e, pl.ds(0, LANES)] = r0 * val
                    buf[e, pl.ds(LANES, LANES)] = r1 * val

        def super_body(sj, c):
            r = row0 + sj * SUPER
            pltpu.sync_copy(src2.at[pl.ds(r, SUPER)], src_v)
            pltpu.sync_copy(dst2.at[pl.ds(r, SUPER)], dst_v)
            pltpu.sync_copy(vals2.at[pl.ds(r, SUPER)], vals_v)

            pltpu.async_copy(x_h.at[src_v.at[0]], bufs[0], sems[0])

            @pl.loop(0, SUPER, step=2)
            def pair_body(j0):
                for b in range(2):
                    j = j0 + b
                    buf, sm, ssm = bufs[b], sems[b], ssems[b]
                    nbuf, nsm, nssm = bufs[1 - b], sems[1 - b], ssems[1 - b]

                    @pl.when((j >= 1) & (j + 1 < SUPER))
                    def _():
                        pltpu.make_async_copy(
                            nbuf, acc.at[dst_v.at[j]], nssm).wait()

                    @pl.when(j + 1 < SUPER)
                    def _():
                        pltpu.async_copy(x_h.at[src_v.at[j + 1]], nbuf, nsm)
                    pltpu.make_async_copy(x_h.at[src_v.at[j]], buf, sm).wait()
                    scale_chunk(j, buf)
                    pltpu.async_copy(buf, acc.at[dst_v.at[j]], ssm, add=True)
            pltpu.make_async_copy(bufs[0], acc.at[dst_v.at[0]], ssems[0]).wait()
            pltpu.make_async_copy(bufs[1], acc.at[dst_v.at[1]], ssems[1]).wait()
            return c
        lax.fori_loop(0, N_SUPER, super_body, 0)
        plsc.subcore_barrier()
        wb = sid * ROWS_PER_TILE
        pltpu.sync_copy(acc.at[pl.ds(wb, ROWS_PER_TILE)],
                        out_h.at[pl.ds(wb, ROWS_PER_TILE)])

    @pl.when(cid == 0)
    def _():
        run_half(xa, outa)

    @pl.when(cid == 1)
    def _():
        run_half(xb, outb)


def _make_spmm():
    mesh = plsc.VectorSubcoreMesh(core_axis_name="c", subcore_axis_name="s",
                                  num_cores=NC, num_subcores=NS)
    return pl.kernel(
        _spmm_body,
        out_type=(jax.ShapeDtypeStruct((N_PAD, HALF), jnp.float32),
                  jax.ShapeDtypeStruct((N_PAD, HALF), jnp.float32)),
        mesh=mesh,
        compiler_params=pltpu.CompilerParams(use_tc_tiling_on_sc=False),
        scratch_types=[
            pltpu.VMEM((SUPER, K_EDGE), jnp.int32),
            pltpu.VMEM((SUPER, K_EDGE), jnp.int32),
            pltpu.VMEM((SUPER, K_EDGE), jnp.float32),
            pltpu.VMEM((K_EDGE, HALF), jnp.float32),
            pltpu.VMEM((K_EDGE, HALF), jnp.float32),
            pltpu.VMEM((ZROWS, HALF), jnp.float32),
            pltpu.VMEM_SHARED((N_PAD, HALF), jnp.float32),
            pltpu.SemaphoreType.DMA,
            pltpu.SemaphoreType.DMA,
            pltpu.SemaphoreType.DMA,
            pltpu.SemaphoreType.DMA,
        ],
    )


def kernel(features, id_embd, adj_indices, adj_values, preference, W1, b1, W2, b2):
    f32 = jnp.float32
    R_MLP = 600
    R_ROW = 1000

    t = pl.pallas_call(
        _mlp_body,
        grid=(NUM_ITEM // R_MLP,),
        in_specs=[
            pl.BlockSpec((R_MLP, DIM_FEAT), lambda i: (i, 0)),
            pl.BlockSpec((DIM_FEAT, 4 * DIM_LATENT), lambda i: (0, 0)),
            pl.BlockSpec((1, 4 * DIM_LATENT), lambda i: (0, 0)),
            pl.BlockSpec((4 * DIM_LATENT, DIM_LATENT), lambda i: (0, 0)),
            pl.BlockSpec((1, DIM_LATENT), lambda i: (0, 0)),
        ],
        out_specs=pl.BlockSpec((R_MLP, DIM_LATENT), lambda i: (i, 0)),
        out_shape=jax.ShapeDtypeStruct((NUM_ITEM, DIM_LATENT), f32),
    )(features, W1, b1.reshape(1, -1), W2, b2.reshape(1, -1))

    temp = jnp.sqrt(jnp.abs((id_embd * id_embd + t * t) * 0.5 + 1e-8))
    x_un = jnp.concatenate([preference, temp], axis=0)

    ega, egb = pl.pallas_call(
        _norm_body,
        grid=(N_NODES // R_ROW,),
        in_specs=[pl.BlockSpec((R_ROW, DIM_LATENT), lambda i: (i, 0))],
        out_specs=[pl.BlockSpec((R_ROW, HALF), lambda i: (i, 0)),
                   pl.BlockSpec((R_ROW, HALF), lambda i: (i, 0))],
        out_shape=[jax.ShapeDtypeStruct((N_NODES, HALF), f32),
                   jax.ShapeDtypeStruct((N_NODES, HALF), f32)],
    )(x_un)

    n_extra = E_PAD - N_EDGES
    pad_idx = (jnp.arange(n_extra, dtype=jnp.int32) * 7) % N_NODES
    dst2 = jnp.concatenate([adj_indices[0], pad_idx]).reshape(E_PAD // K_EDGE, K_EDGE)
    src2 = jnp.concatenate([adj_indices[1], pad_idx]).reshape(E_PAD // K_EDGE, K_EDGE)
    vals2 = jnp.concatenate(
        [adj_values, jnp.zeros((n_extra,), f32)]).reshape(E_PAD // K_EDGE, K_EDGE)

    spmm = _make_spmm()

    s1a, s1b = spmm(ega, egb, src2, dst2, vals2)

    half_spec = pl.BlockSpec((R_ROW, HALF), lambda i: (i, 0))
    l1a, l1b = pl.pallas_call(
        _layer1_body,
        grid=(N_NODES // R_ROW,),
        in_specs=[half_spec] * 4,
        out_specs=[half_spec] * 2,
        out_shape=[jax.ShapeDtypeStruct((N_NODES, HALF), f32)] * 2,
    )(s1a, s1b, ega, egb)

    s2a, s2b = spmm(l1a, l1b, src2, dst2, vals2)

    l2a, l2b = pl.pallas_call(
        _layer2_body,
        grid=(N_NODES // R_ROW,),
        in_specs=[half_spec] * 4,
        out_specs=[half_spec] * 2,
        out_shape=[jax.ShapeDtypeStruct((N_NODES, HALF), f32)] * 2,
    )(s2a, s2b, ega, egb)

    ui = jnp.concatenate([ega + l1a + l2a, egb + l1b + l2b], axis=1)
    return (ui, preference)


# speedup vs baseline: 1.2229x; 1.2229x over previous
import functools

import jax
import jax.numpy as jnp
from jax import lax
from jax.experimental import pallas as pl
from jax.experimental.pallas import tpu as pltpu
from jax.experimental.pallas import tpu_sc as plsc

NUM_USER = 20000
NUM_ITEM = 30000
N_NODES = NUM_USER + NUM_ITEM
DIM_FEAT = 128
DIM_LATENT = 64
HALF = DIM_LATENT // 2
N_EDGES = 800000
NUM_LAYER = 2

NC = 2
NS = 16
LANES = 16

K_EDGE = 128
E_PAD = 819200
CHUNKS_PER_TILE = E_PAD // (NS * K_EDGE)
SUPER = 40
N_SUPER = CHUNKS_PER_TILE // SUPER
N_PAD = 50048
ROWS_PER_TILE = N_PAD // NS
ZROWS = 136


def _mlp_body(f_ref, id_ref, w1_ref, b1_ref, w2_ref, b2_ref, o_ref):
    h = jnp.dot(f_ref[...], w1_ref[...], preferred_element_type=jnp.float32)
    h = h + b1_ref[...]
    h = jnp.where(h >= 0, h, 0.01 * h)
    t = jnp.dot(h, w2_ref[...], preferred_element_type=jnp.float32)
    t = t + b2_ref[...]
    ide = id_ref[...]
    o_ref[...] = jnp.sqrt(jnp.abs((ide * ide + t * t) * 0.5 + 1e-8))


def _norm_body(x_ref, oa_ref, ob_ref):
    x = x_ref[...]
    nrm = jnp.sqrt(jnp.sum(x * x, axis=1, keepdims=True))
    x = x / jnp.maximum(nrm, 1e-12)
    oa_ref[...] = x[:, :HALF]
    ob_ref[...] = x[:, HALF:]


def _cos_weight(sa, sb, ea, eb):
    dot = jnp.sum(sa * ea + sb * eb, axis=1, keepdims=True)
    n1 = jnp.sqrt(jnp.sum(sa * sa + sb * sb, axis=1, keepdims=True))
    n2 = jnp.sqrt(jnp.sum(ea * ea + eb * eb, axis=1, keepdims=True))
    return dot / (jnp.maximum(n1, 1e-8) * jnp.maximum(n2, 1e-8))


def _layer1_body(sa_ref, sb_ref, ea_ref, eb_ref, oa_ref, ob_ref):
    sa, sb = sa_ref[...], sb_ref[...]
    ea, eb = ea_ref[...], eb_ref[...]
    w = _cos_weight(sa, sb, ea, eb)
    oa_ref[...] = w * sa
    ob_ref[...] = w * sb


def _layer2_body(sa_ref, sb_ref, ea_ref, eb_ref, la_ref, lb_ref, o_ref):
    sa, sb = sa_ref[...], sb_ref[...]
    ea, eb = ea_ref[...], eb_ref[...]
    w = _cos_weight(sa, sb, ea, eb)
    o_ref[:, :HALF] = ea + la_ref[...] + w * sa
    o_ref[:, HALF:] = eb + lb_ref[...] + w * sb


def _spmm_body(xa, xb, src2, dst2, vals2, outa, outb,
               src_v, dst_v, vals_v, rows_v, rows_v1, zbuf, acc,
               sem, sem1, ssem0, ssem1):
    cid = lax.axis_index("c")
    sid = lax.axis_index("s")

    def _zrow(i, c):
        z = jnp.zeros((LANES,), jnp.float32)
        zbuf[i, pl.ds(0, LANES)] = z
        zbuf[i, pl.ds(LANES, LANES)] = z
        return c
    lax.fori_loop(0, ZROWS, _zrow, 0)
    base0 = sid * ROWS_PER_TILE
    for i in range(ROWS_PER_TILE // ZROWS):
        pltpu.sync_copy(zbuf, acc.at[pl.ds(base0 + i * ZROWS, ZROWS)])
    plsc.subcore_barrier()
    del base0

    def run_half(x_h, out_h):
        row0 = sid * CHUNKS_PER_TILE
        bufs = (rows_v, rows_v1)
        sems = (sem, sem1)
        ssems = (ssem0, ssem1)

        def scale_chunk(j, buf):
            for g in range(K_EDGE // LANES):
                e0 = g * LANES
                v16 = vals_v[j, pl.ds(e0, LANES)]
                for l in range(LANES):
                    e = e0 + l
                    val = v16[l]
                    r0 = buf[e, pl.ds(0, LANES)]
                    r1 = buf[e, pl.ds(LANES, LANES)]
                    buf[e, pl.ds(0, LANES)] = r0 * val
                    buf[e, pl.ds(LANES, LANES)] = r1 * val

        def super_body(sj, c):
            r = row0 + sj * SUPER
            pltpu.sync_copy(src2.at[pl.ds(r, SUPER)], src_v)
            pltpu.sync_copy(dst2.at[pl.ds(r, SUPER)], dst_v)
            pltpu.sync_copy(vals2.at[pl.ds(r, SUPER)], vals_v)

            pltpu.async_copy(x_h.at[src_v.at[0]], bufs[0], sems[0])

            @pl.loop(0, SUPER, step=2)
            def pair_body(j0):
                for b in range(2):
                    j = j0 + b
                    buf, sm, ssm = bufs[b], sems[b], ssems[b]
                    nbuf, nsm, nssm = bufs[1 - b], sems[1 - b], ssems[1 - b]

                    @pl.when((j >= 1) & (j + 1 < SUPER))
                    def _():
                        pltpu.make_async_copy(
                            nbuf, acc.at[dst_v.at[j]], nssm).wait()

                    @pl.when(j + 1 < SUPER)
                    def _():
                        pltpu.async_copy(x_h.at[src_v.at[j + 1]], nbuf, nsm)
                    pltpu.make_async_copy(x_h.at[src_v.at[j]], buf, sm).wait()
                    scale_chunk(j, buf)
                    pltpu.async_copy(buf, acc.at[dst_v.at[j]], ssm, add=True)
            pltpu.make_async_copy(bufs[0], acc.at[dst_v.at[0]], ssems[0]).wait()
            pltpu.make_async_copy(bufs[1], acc.at[dst_v.at[1]], ssems[1]).wait()
            return c
        lax.fori_loop(0, N_SUPER, super_body, 0)
        plsc.subcore_barrier()
        wb = sid * ROWS_PER_TILE
        pltpu.sync_copy(acc.at[pl.ds(wb, ROWS_PER_TILE)],
                        out_h.at[pl.ds(wb, ROWS_PER_TILE)])

    @pl.when(cid == 0)
    def _():
        run_half(xa, outa)

    @pl.when(cid == 1)
    def _():
        run_half(xb, outb)


def _make_spmm():
    mesh = plsc.VectorSubcoreMesh(core_axis_name="c", subcore_axis_name="s",
                                  num_cores=NC, num_subcores=NS)
    return pl.kernel(
        _spmm_body,
        out_type=(jax.ShapeDtypeStruct((N_PAD, HALF), jnp.float32),
                  jax.ShapeDtypeStruct((N_PAD, HALF), jnp.float32)),
        mesh=mesh,
        compiler_params=pltpu.CompilerParams(use_tc_tiling_on_sc=False),
        scratch_types=[
            pltpu.VMEM((SUPER, K_EDGE), jnp.int32),
            pltpu.VMEM((SUPER, K_EDGE), jnp.int32),
            pltpu.VMEM((SUPER, K_EDGE), jnp.float32),
            pltpu.VMEM((K_EDGE, HALF), jnp.float32),
            pltpu.VMEM((K_EDGE, HALF), jnp.float32),
            pltpu.VMEM((ZROWS, HALF), jnp.float32),
            pltpu.VMEM_SHARED((N_PAD, HALF), jnp.float32),
            pltpu.SemaphoreType.DMA,
            pltpu.SemaphoreType.DMA,
            pltpu.SemaphoreType.DMA,
            pltpu.SemaphoreType.DMA,
        ],
    )


def kernel(features, id_embd, adj_indices, adj_values, preference, W1, b1, W2, b2):
    f32 = jnp.float32
    R_MLP = 600
    R_ROW = 1000

    temp = pl.pallas_call(
        _mlp_body,
        grid=(NUM_ITEM // R_MLP,),
        in_specs=[
            pl.BlockSpec((R_MLP, DIM_FEAT), lambda i: (i, 0)),
            pl.BlockSpec((R_MLP, DIM_LATENT), lambda i: (i, 0)),
            pl.BlockSpec((DIM_FEAT, 4 * DIM_LATENT), lambda i: (0, 0)),
            pl.BlockSpec((1, 4 * DIM_LATENT), lambda i: (0, 0)),
            pl.BlockSpec((4 * DIM_LATENT, DIM_LATENT), lambda i: (0, 0)),
            pl.BlockSpec((1, DIM_LATENT), lambda i: (0, 0)),
        ],
        out_specs=pl.BlockSpec((R_MLP, DIM_LATENT), lambda i: (i, 0)),
        out_shape=jax.ShapeDtypeStruct((NUM_ITEM, DIM_LATENT), f32),
    )(features, id_embd, W1, b1.reshape(1, -1), W2, b2.reshape(1, -1))

    x_un = jnp.concatenate([preference, temp], axis=0)

    ega, egb = pl.pallas_call(
        _norm_body,
        grid=(N_NODES // R_ROW,),
        in_specs=[pl.BlockSpec((R_ROW, DIM_LATENT), lambda i: (i, 0))],
        out_specs=[pl.BlockSpec((R_ROW, HALF), lambda i: (i, 0)),
                   pl.BlockSpec((R_ROW, HALF), lambda i: (i, 0))],
        out_shape=[jax.ShapeDtypeStruct((N_NODES, HALF), f32),
                   jax.ShapeDtypeStruct((N_NODES, HALF), f32)],
    )(x_un)

    n_extra = E_PAD - N_EDGES
    pad_idx = (jnp.arange(n_extra, dtype=jnp.int32) * 7) % N_NODES
    dst2 = jnp.concatenate([adj_indices[0], pad_idx]).reshape(E_PAD // K_EDGE, K_EDGE)
    src2 = jnp.concatenate([adj_indices[1], pad_idx]).reshape(E_PAD // K_EDGE, K_EDGE)
    vals2 = jnp.concatenate(
        [adj_values, jnp.zeros((n_extra,), f32)]).reshape(E_PAD // K_EDGE, K_EDGE)

    spmm = _make_spmm()

    s1a, s1b = spmm(ega, egb, src2, dst2, vals2)

    half_spec = pl.BlockSpec((R_ROW, HALF), lambda i: (i, 0))
    l1a, l1b = pl.pallas_call(
        _layer1_body,
        grid=(N_NODES // R_ROW,),
        in_specs=[half_spec] * 4,
        out_specs=[half_spec] * 2,
        out_shape=[jax.ShapeDtypeStruct((N_NODES, HALF), f32)] * 2,
    )(s1a, s1b, ega, egb)

    s2a, s2b = spmm(l1a, l1b, src2, dst2, vals2)

    ui = pl.pallas_call(
        _layer2_body,
        grid=(N_NODES // R_ROW,),
        in_specs=[half_spec] * 6,
        out_specs=pl.BlockSpec((R_ROW, DIM_LATENT), lambda i: (i, 0)),
        out_shape=jax.ShapeDtypeStruct((N_NODES, DIM_LATENT), f32),
    )(s2a, s2b, ega, egb, l1a, l1b)

    return (ui, preference)


# speedup vs baseline: 1.2714x; 1.0396x over previous
import functools

import jax
import jax.numpy as jnp
from jax import lax
from jax.experimental import pallas as pl
from jax.experimental.pallas import tpu as pltpu
from jax.experimental.pallas import tpu_sc as plsc

NUM_USER = 20000
NUM_ITEM = 30000
N_NODES = NUM_USER + NUM_ITEM
DIM_FEAT = 128
DIM_LATENT = 64
HALF = DIM_LATENT // 2
N_EDGES = 800000
NUM_LAYER = 2

NC = 2
NS = 16
LANES = 16

K_EDGE = 128
E_PAD = 819200
CHUNKS_PER_TILE = E_PAD // (NS * K_EDGE)
SUPER = 16
N_SUPER = CHUNKS_PER_TILE // SUPER
DEPTH = 4
N_PAD = 50048
ROWS_PER_TILE = N_PAD // NS
ZROWS = 136


def _mlp_body(f_ref, id_ref, w1_ref, b1_ref, w2_ref, b2_ref, o_ref):
    h = jnp.dot(f_ref[...], w1_ref[...], preferred_element_type=jnp.float32)
    h = h + b1_ref[...]
    h = jnp.where(h >= 0, h, 0.01 * h)
    t = jnp.dot(h, w2_ref[...], preferred_element_type=jnp.float32)
    t = t + b2_ref[...]
    ide = id_ref[...]
    o_ref[...] = jnp.sqrt(jnp.abs((ide * ide + t * t) * 0.5 + 1e-8))


def _norm_body(x_ref, oa_ref, ob_ref):
    x = x_ref[...]
    nrm = jnp.sqrt(jnp.sum(x * x, axis=1, keepdims=True))
    x = x / jnp.maximum(nrm, 1e-12)
    oa_ref[...] = x[:, :HALF]
    ob_ref[...] = x[:, HALF:]


def _cos_weight(sa, sb, ea, eb):
    dot = jnp.sum(sa * ea + sb * eb, axis=1, keepdims=True)
    n1 = jnp.sqrt(jnp.sum(sa * sa + sb * sb, axis=1, keepdims=True))
    n2 = jnp.sqrt(jnp.sum(ea * ea + eb * eb, axis=1, keepdims=True))
    return dot / (jnp.maximum(n1, 1e-8) * jnp.maximum(n2, 1e-8))


def _layer1_body(sa_ref, sb_ref, ea_ref, eb_ref, oa_ref, ob_ref):
    sa, sb = sa_ref[...], sb_ref[...]
    ea, eb = ea_ref[...], eb_ref[...]
    w = _cos_weight(sa, sb, ea, eb)
    oa_ref[...] = w * sa
    ob_ref[...] = w * sb


def _layer2_body(sa_ref, sb_ref, ea_ref, eb_ref, la_ref, lb_ref, o_ref):
    sa, sb = sa_ref[...], sb_ref[...]
    ea, eb = ea_ref[...], eb_ref[...]
    w = _cos_weight(sa, sb, ea, eb)
    o_ref[:, :HALF] = ea + la_ref[...] + w * sa
    o_ref[:, HALF:] = eb + lb_ref[...] + w * sb


def _spmm_body(xa, xb, src2, dst2, vals2, outa, outb,
               src_v, dst_v, vals_v, rows_v, rows_v1, rows_v2, rows_v3,
               zbuf, acc, sem, sem1, sem2, sem3, ssem0, ssem1, ssem2, ssem3):
    cid = lax.axis_index("c")
    sid = lax.axis_index("s")

    def _zrow(i, c):
        z = jnp.zeros((LANES,), jnp.float32)
        zbuf[i, pl.ds(0, LANES)] = z
        zbuf[i, pl.ds(LANES, LANES)] = z
        return c
    lax.fori_loop(0, ZROWS, _zrow, 0)
    base0 = sid * ROWS_PER_TILE
    for i in range(ROWS_PER_TILE // ZROWS):
        pltpu.sync_copy(zbuf, acc.at[pl.ds(base0 + i * ZROWS, ZROWS)])
    plsc.subcore_barrier()
    del base0

    def run_half(x_h, out_h):
        row0 = sid * CHUNKS_PER_TILE
        bufs = (rows_v, rows_v1, rows_v2, rows_v3)
        sems = (sem, sem1, sem2, sem3)
        ssems = (ssem0, ssem1, ssem2, ssem3)

        def scale_chunk(j, buf):
            for g in range(K_EDGE // LANES):
                e0 = g * LANES
                v16 = vals_v[j, pl.ds(e0, LANES)]
                for l in range(LANES):
                    e = e0 + l
                    val = v16[l]
                    r0 = buf[e, pl.ds(0, LANES)]
                    r1 = buf[e, pl.ds(LANES, LANES)]
                    buf[e, pl.ds(0, LANES)] = r0 * val
                    buf[e, pl.ds(LANES, LANES)] = r1 * val

        def super_body(sj, c):
            r = row0 + sj * SUPER
            pltpu.sync_copy(src2.at[pl.ds(r, SUPER)], src_v)
            pltpu.sync_copy(dst2.at[pl.ds(r, SUPER)], dst_v)
            pltpu.sync_copy(vals2.at[pl.ds(r, SUPER)], vals_v)

            for p in range(DEPTH - 1):
                pltpu.async_copy(x_h.at[src_v.at[p]], bufs[p], sems[p])

            @pl.loop(0, SUPER, step=DEPTH)
            def quad_body(j0):
                for b in range(DEPTH):
                    j = j0 + b
                    buf, sm, ssm = bufs[b], sems[b], ssems[b]
                    nj = j + DEPTH - 1
                    nb = (b + DEPTH - 1) % DEPTH

                    @pl.when((j >= 1) & (nj < SUPER))
                    def _():
                        pltpu.make_async_copy(
                            bufs[nb], acc.at[dst_v.at[j]], ssems[nb]).wait()

                    @pl.when(nj < SUPER)
                    def _():
                        pltpu.async_copy(x_h.at[src_v.at[nj]], bufs[nb], sems[nb])
                    pltpu.make_async_copy(x_h.at[src_v.at[j]], buf, sm).wait()
                    scale_chunk(j, buf)
                    pltpu.async_copy(buf, acc.at[dst_v.at[j]], ssm, add=True)
            for p in range(DEPTH):
                pltpu.make_async_copy(bufs[p], acc.at[dst_v.at[p]], ssems[p]).wait()
            return c
        lax.fori_loop(0, N_SUPER, super_body, 0)
        plsc.subcore_barrier()
        wb = sid * ROWS_PER_TILE
        pltpu.sync_copy(acc.at[pl.ds(wb, ROWS_PER_TILE)],
                        out_h.at[pl.ds(wb, ROWS_PER_TILE)])

    @pl.when(cid == 0)
    def _():
        run_half(xa, outa)

    @pl.when(cid == 1)
    def _():
        run_half(xb, outb)


def _make_spmm():
    mesh = plsc.VectorSubcoreMesh(core_axis_name="c", subcore_axis_name="s",
                                  num_cores=NC, num_subcores=NS)
    return pl.kernel(
        _spmm_body,
        out_type=(jax.ShapeDtypeStruct((N_PAD, HALF), jnp.float32),
                  jax.ShapeDtypeStruct((N_PAD, HALF), jnp.float32)),
        mesh=mesh,
        compiler_params=pltpu.CompilerParams(use_tc_tiling_on_sc=False),
        scratch_types=[
            pltpu.VMEM((SUPER, K_EDGE), jnp.int32),
            pltpu.VMEM((SUPER, K_EDGE), jnp.int32),
            pltpu.VMEM((SUPER, K_EDGE), jnp.float32),
            pltpu.VMEM((K_EDGE, HALF), jnp.float32),
            pltpu.VMEM((K_EDGE, HALF), jnp.float32),
            pltpu.VMEM((K_EDGE, HALF), jnp.float32),
            pltpu.VMEM((K_EDGE, HALF), jnp.float32),
            pltpu.VMEM((ZROWS, HALF), jnp.float32),
            pltpu.VMEM_SHARED((N_PAD, HALF), jnp.float32),
        ] + [pltpu.SemaphoreType.DMA] * 8,
    )


def kernel(features, id_embd, adj_indices, adj_values, preference, W1, b1, W2, b2):
    f32 = jnp.float32
    R_MLP = 600
    R_ROW = 1000

    temp = pl.pallas_call(
        _mlp_body,
        grid=(NUM_ITEM // R_MLP,),
        in_specs=[
            pl.BlockSpec((R_MLP, DIM_FEAT), lambda i: (i, 0)),
            pl.BlockSpec((R_MLP, DIM_LATENT), lambda i: (i, 0)),
            pl.BlockSpec((DIM_FEAT, 4 * DIM_LATENT), lambda i: (0, 0)),
            pl.BlockSpec((1, 4 * DIM_LATENT), lambda i: (0, 0)),
            pl.BlockSpec((4 * DIM_LATENT, DIM_LATENT), lambda i: (0, 0)),
            pl.BlockSpec((1, DIM_LATENT), lambda i: (0, 0)),
        ],
        out_specs=pl.BlockSpec((R_MLP, DIM_LATENT), lambda i: (i, 0)),
        out_shape=jax.ShapeDtypeStruct((NUM_ITEM, DIM_LATENT), f32),
    )(features, id_embd, W1, b1.reshape(1, -1), W2, b2.reshape(1, -1))

    x_un = jnp.concatenate([preference, temp], axis=0)

    ega, egb = pl.pallas_call(
        _norm_body,
        grid=(N_NODES // R_ROW,),
        in_specs=[pl.BlockSpec((R_ROW, DIM_LATENT), lambda i: (i, 0))],
        out_specs=[pl.BlockSpec((R_ROW, HALF), lambda i: (i, 0)),
                   pl.BlockSpec((R_ROW, HALF), lambda i: (i, 0))],
        out_shape=[jax.ShapeDtypeStruct((N_NODES, HALF), f32),
                   jax.ShapeDtypeStruct((N_NODES, HALF), f32)],
    )(x_un)

    n_extra = E_PAD - N_EDGES
    pad_idx = (jnp.arange(n_extra, dtype=jnp.int32) * 7) % N_NODES
    dst2 = jnp.concatenate([adj_indices[0], pad_idx]).reshape(E_PAD // K_EDGE, K_EDGE)
    src2 = jnp.concatenate([adj_indices[1], pad_idx]).reshape(E_PAD // K_EDGE, K_EDGE)
    vals2 = jnp.concatenate(
        [adj_values, jnp.zeros((n_extra,), f32)]).reshape(E_PAD // K_EDGE, K_EDGE)

    spmm = _make_spmm()

    s1a, s1b = spmm(ega, egb, src2, dst2, vals2)

    half_spec = pl.BlockSpec((R_ROW, HALF), lambda i: (i, 0))
    l1a, l1b = pl.pallas_call(
        _layer1_body,
        grid=(N_NODES // R_ROW,),
        in_specs=[half_spec] * 4,
        out_specs=[half_spec] * 2,
        out_shape=[jax.ShapeDtypeStruct((N_NODES, HALF), f32)] * 2,
    )(s1a, s1b, ega, egb)

    s2a, s2b = spmm(l1a, l1b, src2, dst2, vals2)

    ui = pl.pallas_call(
        _layer2_body,
        grid=(N_NODES // R_ROW,),
        in_specs=[half_spec] * 6,
        out_specs=pl.BlockSpec((R_ROW, DIM_LATENT), lambda i: (i, 0)),
        out_shape=jax.ShapeDtypeStruct((N_NODES, DIM_LATENT), f32),
    )(s2a, s2b, ega, egb, l1a, l1b)

    return (ui, preference)


# speedup vs baseline: 1.3354x; 1.0504x over previous
import functools

import jax
import jax.numpy as jnp
from jax import lax
from jax.experimental import pallas as pl
from jax.experimental.pallas import tpu as pltpu
from jax.experimental.pallas import tpu_sc as plsc

NUM_USER = 20000
NUM_ITEM = 30000
N_NODES = NUM_USER + NUM_ITEM
DIM_FEAT = 128
DIM_LATENT = 64
HALF = DIM_LATENT // 2
N_EDGES = 800000
NUM_LAYER = 2

NC = 2
NS = 16
LANES = 16

K_EDGE = 128
E_PAD = 819200
CHUNKS_PER_TILE = E_PAD // (NS * K_EDGE)
SUPER = 16
N_SUPER = CHUNKS_PER_TILE // SUPER
DEPTH = 4
N_PAD = 50048
ROWS_PER_TILE = N_PAD // NS
ZROWS = 136


def _mlp_body(f_ref, id_ref, w1_ref, b1_ref, w2_ref, b2_ref, o_ref):
    h = jnp.dot(f_ref[...], w1_ref[...], preferred_element_type=jnp.float32)
    h = h + b1_ref[...]
    h = jnp.where(h >= 0, h, 0.01 * h)
    t = jnp.dot(h, w2_ref[...], preferred_element_type=jnp.float32)
    t = t + b2_ref[...]
    ide = id_ref[...]
    o_ref[...] = jnp.sqrt(jnp.abs((ide * ide + t * t) * 0.5 + 1e-8))


def _norm_body(x_ref, oa_ref, ob_ref):
    x = x_ref[...]
    nrm = jnp.sqrt(jnp.sum(x * x, axis=1, keepdims=True))
    x = x / jnp.maximum(nrm, 1e-12)
    oa_ref[...] = x[:, :HALF]
    ob_ref[...] = x[:, HALF:]


def _cos_weight(sa, sb, ea, eb):
    dot = jnp.sum(sa * ea + sb * eb, axis=1, keepdims=True)
    n1 = jnp.sqrt(jnp.sum(sa * sa + sb * sb, axis=1, keepdims=True))
    n2 = jnp.sqrt(jnp.sum(ea * ea + eb * eb, axis=1, keepdims=True))
    return dot / (jnp.maximum(n1, 1e-8) * jnp.maximum(n2, 1e-8))


def _layer1_body(sa_ref, sb_ref, ea_ref, eb_ref, oa_ref, ob_ref):
    sa, sb = sa_ref[...], sb_ref[...]
    ea, eb = ea_ref[...], eb_ref[...]
    w = _cos_weight(sa, sb, ea, eb)
    oa_ref[...] = w * sa
    ob_ref[...] = w * sb


def _layer2_body(sa_ref, sb_ref, ea_ref, eb_ref, la_ref, lb_ref, o_ref):
    sa, sb = sa_ref[...], sb_ref[...]
    ea, eb = ea_ref[...], eb_ref[...]
    w = _cos_weight(sa, sb, ea, eb)
    o_ref[:, :HALF] = ea + la_ref[...] + w * sa
    o_ref[:, HALF:] = eb + lb_ref[...] + w * sb


def _spmm_body(xa, xb, src2, dst2, vals2, outa, outb,
               src_v, dst_v, vals_v, rows_v, rows_v1, rows_v2, rows_v3,
               zbuf, acc, sem, sem1, sem2, sem3, ssem0, ssem1, ssem2, ssem3):
    cid = lax.axis_index("c")
    sid = lax.axis_index("s")

    def _zrow(i, c):
        z = jnp.zeros((LANES,), jnp.float32)
        zbuf[i, pl.ds(0, LANES)] = z
        zbuf[i, pl.ds(LANES, LANES)] = z
        return c
    lax.fori_loop(0, ZROWS, _zrow, 0)
    base0 = sid * ROWS_PER_TILE
    for i in range(ROWS_PER_TILE // ZROWS):
        pltpu.sync_copy(zbuf, acc.at[pl.ds(base0 + i * ZROWS, ZROWS)])
    plsc.subcore_barrier()
    del base0

    def run_half(x_h, out_h):
        row0 = sid * CHUNKS_PER_TILE
        bufs = (rows_v, rows_v1, rows_v2, rows_v3)
        sems = (sem, sem1, sem2, sem3)
        ssems = (ssem0, ssem1, ssem2, ssem3)

        def scale_chunk(j, buf):
            @plsc.parallel_loop(0, K_EDGE // LANES, unroll=2)
            def _(g):
                e0 = g * LANES
                v16 = vals_v[j, pl.ds(e0, LANES)]
                for l in range(LANES):
                    e = e0 + l
                    val = v16[l]
                    r0 = buf[e, pl.ds(0, LANES)]
                    r1 = buf[e, pl.ds(LANES, LANES)]
                    buf[e, pl.ds(0, LANES)] = r0 * val
                    buf[e, pl.ds(LANES, LANES)] = r1 * val

        def super_body(sj, c):
            r = row0 + sj * SUPER
            pltpu.sync_copy(src2.at[pl.ds(r, SUPER)], src_v)
            pltpu.sync_copy(dst2.at[pl.ds(r, SUPER)], dst_v)
            pltpu.sync_copy(vals2.at[pl.ds(r, SUPER)], vals_v)

            for p in range(DEPTH - 1):
                pltpu.async_copy(x_h.at[src_v.at[p]], bufs[p], sems[p])

            @pl.loop(0, SUPER, step=DEPTH)
            def quad_body(j0):
                for b in range(DEPTH):
                    j = j0 + b
                    buf, sm, ssm = bufs[b], sems[b], ssems[b]
                    nj = j + DEPTH - 1
                    nb = (b + DEPTH - 1) % DEPTH

                    @pl.when((j >= 1) & (nj < SUPER))
                    def _():
                        pltpu.make_async_copy(
                            bufs[nb], acc.at[dst_v.at[j]], ssems[nb]).wait()

                    @pl.when(nj < SUPER)
                    def _():
                        pltpu.async_copy(x_h.at[src_v.at[nj]], bufs[nb], sems[nb])
                    pltpu.make_async_copy(x_h.at[src_v.at[j]], buf, sm).wait()
                    scale_chunk(j, buf)
                    pltpu.async_copy(buf, acc.at[dst_v.at[j]], ssm, add=True)
            for p in range(DEPTH):
                pltpu.make_async_copy(bufs[p], acc.at[dst_v.at[p]], ssems[p]).wait()
            return c
        lax.fori_loop(0, N_SUPER, super_body, 0)
        plsc.subcore_barrier()
        wb = sid * ROWS_PER_TILE
        pltpu.sync_copy(acc.at[pl.ds(wb, ROWS_PER_TILE)],
                        out_h.at[pl.ds(wb, ROWS_PER_TILE)])

    @pl.when(cid == 0)
    def _():
        run_half(xa, outa)

    @pl.when(cid == 1)
    def _():
        run_half(xb, outb)


def _make_spmm():
    mesh = plsc.VectorSubcoreMesh(core_axis_name="c", subcore_axis_name="s",
                                  num_cores=NC, num_subcores=NS)
    return pl.kernel(
        _spmm_body,
        out_type=(jax.ShapeDtypeStruct((N_PAD, HALF), jnp.float32),
                  jax.ShapeDtypeStruct((N_PAD, HALF), jnp.float32)),
        mesh=mesh,
        compiler_params=pltpu.CompilerParams(use_tc_tiling_on_sc=False),
        scratch_types=[
            pltpu.VMEM((SUPER, K_EDGE), jnp.int32),
            pltpu.VMEM((SUPER, K_EDGE), jnp.int32),
            pltpu.VMEM((SUPER, K_EDGE), jnp.float32),
            pltpu.VMEM((K_EDGE, HALF), jnp.float32),
            pltpu.VMEM((K_EDGE, HALF), jnp.float32),
            pltpu.VMEM((K_EDGE, HALF), jnp.float32),
            pltpu.VMEM((K_EDGE, HALF), jnp.float32),
            pltpu.VMEM((ZROWS, HALF), jnp.float32),
            pltpu.VMEM_SHARED((N_PAD, HALF), jnp.float32),
        ] + [pltpu.SemaphoreType.DMA] * 8,
    )


def kernel(features, id_embd, adj_indices, adj_values, preference, W1, b1, W2, b2):
    f32 = jnp.float32
    R_MLP = 600
    R_ROW = 1000

    temp = pl.pallas_call(
        _mlp_body,
        grid=(NUM_ITEM // R_MLP,),
        in_specs=[
            pl.BlockSpec((R_MLP, DIM_FEAT), lambda i: (i, 0)),
            pl.BlockSpec((R_MLP, DIM_LATENT), lambda i: (i, 0)),
            pl.BlockSpec((DIM_FEAT, 4 * DIM_LATENT), lambda i: (0, 0)),
            pl.BlockSpec((1, 4 * DIM_LATENT), lambda i: (0, 0)),
            pl.BlockSpec((4 * DIM_LATENT, DIM_LATENT), lambda i: (0, 0)),
            pl.BlockSpec((1, DIM_LATENT), lambda i: (0, 0)),
        ],
        out_specs=pl.BlockSpec((R_MLP, DIM_LATENT), lambda i: (i, 0)),
        out_shape=jax.ShapeDtypeStruct((NUM_ITEM, DIM_LATENT), f32),
    )(features, id_embd, W1, b1.reshape(1, -1), W2, b2.reshape(1, -1))

    x_un = jnp.concatenate([preference, temp], axis=0)

    ega, egb = pl.pallas_call(
        _norm_body,
        grid=(N_NODES // R_ROW,),
        in_specs=[pl.BlockSpec((R_ROW, DIM_LATENT), lambda i: (i, 0))],
        out_specs=[pl.BlockSpec((R_ROW, HALF), lambda i: (i, 0)),
                   pl.BlockSpec((R_ROW, HALF), lambda i: (i, 0))],
        out_shape=[jax.ShapeDtypeStruct((N_NODES, HALF), f32),
                   jax.ShapeDtypeStruct((N_NODES, HALF), f32)],
    )(x_un)

    n_extra = E_PAD - N_EDGES
    pad_idx = (jnp.arange(n_extra, dtype=jnp.int32) * 7) % N_NODES
    dst2 = jnp.concatenate([adj_indices[0], pad_idx]).reshape(E_PAD // K_EDGE, K_EDGE)
    src2 = jnp.concatenate([adj_indices[1], pad_idx]).reshape(E_PAD // K_EDGE, K_EDGE)
    vals2 = jnp.concatenate(
        [adj_values, jnp.zeros((n_extra,), f32)]).reshape(E_PAD // K_EDGE, K_EDGE)

    spmm = _make_spmm()

    s1a, s1b = spmm(ega, egb, src2, dst2, vals2)

    half_spec = pl.BlockSpec((R_ROW, HALF), lambda i: (i, 0))
    l1a, l1b = pl.pallas_call(
        _layer1_body,
        grid=(N_NODES // R_ROW,),
        in_specs=[half_spec] * 4,
        out_specs=[half_spec] * 2,
        out_shape=[jax.ShapeDtypeStruct((N_NODES, HALF), f32)] * 2,
    )(s1a, s1b, ega, egb)

    s2a, s2b = spmm(l1a, l1b, src2, dst2, vals2)

    ui = pl.pallas_call(
        _layer2_body,
        grid=(N_NODES // R_ROW,),
        in_specs=[half_spec] * 6,
        out_specs=pl.BlockSpec((R_ROW, DIM_LATENT), lambda i: (i, 0)),
        out_shape=jax.ShapeDtypeStruct((N_NODES, DIM_LATENT), f32),
    )(s2a, s2b, ega, egb, l1a, l1b)

    return (ui, preference)


# speedup vs baseline: 1.3357x; 1.0003x over previous
import functools

import jax
import jax.numpy as jnp
from jax import lax
from jax.experimental import pallas as pl
from jax.experimental.pallas import tpu as pltpu
from jax.experimental.pallas import tpu_sc as plsc

NUM_USER = 20000
NUM_ITEM = 30000
N_NODES = NUM_USER + NUM_ITEM
DIM_FEAT = 128
DIM_LATENT = 64
HALF = DIM_LATENT // 2
N_EDGES = 800000
NUM_LAYER = 2

NC = 2
NS = 16
LANES = 16

K_EDGE = 128
E_PAD = 819200
CHUNKS_PER_TILE = E_PAD // (NS * K_EDGE)
SUPER = 16
N_SUPER = CHUNKS_PER_TILE // SUPER
DEPTH = 4
N_PAD = 50048
ROWS_PER_TILE = N_PAD // NS
ZROWS = 136


def _mlp_body(f_ref, id_ref, w1_ref, b1_ref, w2_ref, b2_ref, o_ref):
    h = jnp.dot(f_ref[...], w1_ref[...], preferred_element_type=jnp.float32)
    h = h + b1_ref[...]
    h = jnp.where(h >= 0, h, 0.01 * h)
    t = jnp.dot(h, w2_ref[...], preferred_element_type=jnp.float32)
    t = t + b2_ref[...]
    ide = id_ref[...]
    o_ref[...] = jnp.sqrt(jnp.abs((ide * ide + t * t) * 0.5 + 1e-8))


def _norm_body(x_ref, oa_ref, ob_ref):
    x = x_ref[...]
    nrm = jnp.sqrt(jnp.sum(x * x, axis=1, keepdims=True))
    x = x / jnp.maximum(nrm, 1e-12)
    oa_ref[...] = x[:, :HALF]
    ob_ref[...] = x[:, HALF:]


def _cos_weight(sa, sb, ea, eb):
    dot = jnp.sum(sa * ea + sb * eb, axis=1, keepdims=True)
    n1 = jnp.sqrt(jnp.sum(sa * sa + sb * sb, axis=1, keepdims=True))
    n2 = jnp.sqrt(jnp.sum(ea * ea + eb * eb, axis=1, keepdims=True))
    return dot / (jnp.maximum(n1, 1e-8) * jnp.maximum(n2, 1e-8))


def _layer1_body(sa_ref, sb_ref, ea_ref, eb_ref, oa_ref, ob_ref):
    sa, sb = sa_ref[...], sb_ref[...]
    ea, eb = ea_ref[...], eb_ref[...]
    w = _cos_weight(sa, sb, ea, eb)
    oa_ref[...] = w * sa
    ob_ref[...] = w * sb


def _layer2_body(sa_ref, sb_ref, ea_ref, eb_ref, la_ref, lb_ref, o_ref):
    sa, sb = sa_ref[...], sb_ref[...]
    ea, eb = ea_ref[...], eb_ref[...]
    w = _cos_weight(sa, sb, ea, eb)
    o_ref[:, :HALF] = ea + la_ref[...] + w * sa
    o_ref[:, HALF:] = eb + lb_ref[...] + w * sb


def _spmm_body(xa, xb, src2, dst2, vals2, outa, outb,
               src_v, dst_v, vals_v, rows_v, rows_v1, rows_v2, rows_v3,
               zbuf, acc, sem, sem1, sem2, sem3, ssem0, ssem1, ssem2, ssem3):
    cid = lax.axis_index("c")
    sid = lax.axis_index("s")

    def _zrow(i, c):
        z = jnp.zeros((LANES,), jnp.float32)
        zbuf[i, pl.ds(0, LANES)] = z
        zbuf[i, pl.ds(LANES, LANES)] = z
        return c
    lax.fori_loop(0, ZROWS, _zrow, 0)
    base0 = sid * ROWS_PER_TILE
    for i in range(ROWS_PER_TILE // ZROWS):
        pltpu.sync_copy(zbuf, acc.at[pl.ds(base0 + i * ZROWS, ZROWS)])
    plsc.subcore_barrier()
    del base0

    def run_half(x_h, out_h):
        row0 = sid * CHUNKS_PER_TILE
        bufs = (rows_v, rows_v1, rows_v2, rows_v3)
        sems = (sem, sem1, sem2, sem3)
        ssems = (ssem0, ssem1, ssem2, ssem3)

        def scale_chunk(j, buf):
            @plsc.parallel_loop(0, K_EDGE // LANES, unroll=4)
            def _(g):
                e0 = g * LANES
                v16 = vals_v[j, pl.ds(e0, LANES)]
                for l in range(LANES):
                    e = e0 + l
                    val = v16[l]
                    r0 = buf[e, pl.ds(0, LANES)]
                    r1 = buf[e, pl.ds(LANES, LANES)]
                    buf[e, pl.ds(0, LANES)] = r0 * val
                    buf[e, pl.ds(LANES, LANES)] = r1 * val

        def super_body(sj, c):
            r = row0 + sj * SUPER
            pltpu.sync_copy(src2.at[pl.ds(r, SUPER)], src_v)
            pltpu.sync_copy(dst2.at[pl.ds(r, SUPER)], dst_v)
            pltpu.sync_copy(vals2.at[pl.ds(r, SUPER)], vals_v)

            for p in range(DEPTH - 1):
                pltpu.async_copy(x_h.at[src_v.at[p]], bufs[p], sems[p])

            @pl.loop(0, SUPER, step=DEPTH)
            def quad_body(j0):
                for b in range(DEPTH):
                    j = j0 + b
                    buf, sm, ssm = bufs[b], sems[b], ssems[b]
                    nj = j + DEPTH - 1
                    nb = (b + DEPTH - 1) % DEPTH

                    @pl.when((j >= 1) & (nj < SUPER))
                    def _():
                        pltpu.make_async_copy(
                            bufs[nb], acc.at[dst_v.at[j]], ssems[nb]).wait()

                    @pl.when(nj < SUPER)
                    def _():
                        pltpu.async_copy(x_h.at[src_v.at[nj]], bufs[nb], sems[nb])
                    pltpu.make_async_copy(x_h.at[src_v.at[j]], buf, sm).wait()
                    scale_chunk(j, buf)
                    pltpu.async_copy(buf, acc.at[dst_v.at[j]], ssm, add=True)
            for p in range(DEPTH):
                pltpu.make_async_copy(bufs[p], acc.at[dst_v.at[p]], ssems[p]).wait()
            return c
        lax.fori_loop(0, N_SUPER, super_body, 0)
        plsc.subcore_barrier()
        wb = sid * ROWS_PER_TILE
        pltpu.sync_copy(acc.at[pl.ds(wb, ROWS_PER_TILE)],
                        out_h.at[pl.ds(wb, ROWS_PER_TILE)])

    @pl.when(cid == 0)
    def _():
        run_half(xa, outa)

    @pl.when(cid == 1)
    def _():
        run_half(xb, outb)


def _make_spmm():
    mesh = plsc.VectorSubcoreMesh(core_axis_name="c", subcore_axis_name="s",
                                  num_cores=NC, num_subcores=NS)
    return pl.kernel(
        _spmm_body,
        out_type=(jax.ShapeDtypeStruct((N_PAD, HALF), jnp.float32),
                  jax.ShapeDtypeStruct((N_PAD, HALF), jnp.float32)),
        mesh=mesh,
        compiler_params=pltpu.CompilerParams(use_tc_tiling_on_sc=False),
        scratch_types=[
            pltpu.VMEM((SUPER, K_EDGE), jnp.int32),
            pltpu.VMEM((SUPER, K_EDGE), jnp.int32),
            pltpu.VMEM((SUPER, K_EDGE), jnp.float32),
            pltpu.VMEM((K_EDGE, HALF), jnp.float32),
            pltpu.VMEM((K_EDGE, HALF), jnp.float32),
            pltpu.VMEM((K_EDGE, HALF), jnp.float32),
            pltpu.VMEM((K_EDGE, HALF), jnp.float32),
            pltpu.VMEM((ZROWS, HALF), jnp.float32),
            pltpu.VMEM_SHARED((N_PAD, HALF), jnp.float32),
        ] + [pltpu.SemaphoreType.DMA] * 8,
    )


def kernel(features, id_embd, adj_indices, adj_values, preference, W1, b1, W2, b2):
    f32 = jnp.float32
    R_MLP = 600
    R_ROW = 1000

    temp = pl.pallas_call(
        _mlp_body,
        grid=(NUM_ITEM // R_MLP,),
        in_specs=[
            pl.BlockSpec((R_MLP, DIM_FEAT), lambda i: (i, 0)),
            pl.BlockSpec((R_MLP, DIM_LATENT), lambda i: (i, 0)),
            pl.BlockSpec((DIM_FEAT, 4 * DIM_LATENT), lambda i: (0, 0)),
            pl.BlockSpec((1, 4 * DIM_LATENT), lambda i: (0, 0)),
            pl.BlockSpec((4 * DIM_LATENT, DIM_LATENT), lambda i: (0, 0)),
            pl.BlockSpec((1, DIM_LATENT), lambda i: (0, 0)),
        ],
        out_specs=pl.BlockSpec((R_MLP, DIM_LATENT), lambda i: (i, 0)),
        out_shape=jax.ShapeDtypeStruct((NUM_ITEM, DIM_LATENT), f32),
    )(features, id_embd, W1, b1.reshape(1, -1), W2, b2.reshape(1, -1))

    x_un = jnp.concatenate([preference, temp], axis=0)

    ega, egb = pl.pallas_call(
        _norm_body,
        grid=(N_NODES // R_ROW,),
        in_specs=[pl.BlockSpec((R_ROW, DIM_LATENT), lambda i: (i, 0))],
        out_specs=[pl.BlockSpec((R_ROW, HALF), lambda i: (i, 0)),
                   pl.BlockSpec((R_ROW, HALF), lambda i: (i, 0))],
        out_shape=[jax.ShapeDtypeStruct((N_NODES, HALF), f32),
                   jax.ShapeDtypeStruct((N_NODES, HALF), f32)],
    )(x_un)

    n_extra = E_PAD - N_EDGES
    pad_idx = (jnp.arange(n_extra, dtype=jnp.int32) * 7) % N_NODES
    dst2 = jnp.concatenate([adj_indices[0], pad_idx]).reshape(E_PAD // K_EDGE, K_EDGE)
    src2 = jnp.concatenate([adj_indices[1], pad_idx]).reshape(E_PAD // K_EDGE, K_EDGE)
    vals2 = jnp.concatenate(
        [adj_values, jnp.zeros((n_extra,), f32)]).reshape(E_PAD // K_EDGE, K_EDGE)

    spmm = _make_spmm()

    s1a, s1b = spmm(ega, egb, src2, dst2, vals2)

    half_spec = pl.BlockSpec((R_ROW, HALF), lambda i: (i, 0))
    l1a, l1b = pl.pallas_call(
        _layer1_body,
        grid=(N_NODES // R_ROW,),
        in_specs=[half_spec] * 4,
        out_specs=[half_spec] * 2,
        out_shape=[jax.ShapeDtypeStruct((N_NODES, HALF), f32)] * 2,
    )(s1a, s1b, ega, egb)

    s2a, s2b = spmm(l1a, l1b, src2, dst2, vals2)

    ui = pl.pallas_call(
        _layer2_body,
        grid=(N_NODES // R_ROW,),
        in_specs=[half_spec] * 6,
        out_specs=pl.BlockSpec((R_ROW, DIM_LATENT), lambda i: (i, 0)),
        out_shape=jax.ShapeDtypeStruct((N_NODES, DIM_LATENT), f32),
    )(s2a, s2b, ega, egb, l1a, l1b)

    return (ui, preference)
